# speedup vs baseline: 4.9745x; 4.9745x over previous
import functools

import jax
import jax.numpy as jnp
from jax import lax
from jax.experimental import pallas as pl
from jax.experimental.pallas import tpu as pltpu
from jax.experimental.pallas import tpu_sc as plsc

EMBED_DIM = 128
CHUNK = 128
NUM_WORKERS = 32


@functools.partial(jax.jit, static_argnames=("n_chunks",))
def _sc_gather(weight, idx_grp, n_chunks):
    batch = NUM_WORKERS * n_chunks * CHUNK
    mesh = plsc.VectorSubcoreMesh(core_axis_name="c", subcore_axis_name="s")

    @functools.partial(
        pl.kernel,
        mesh=mesh,
        out_type=jax.ShapeDtypeStruct((batch, EMBED_DIM), jnp.float32),
        scratch_types=[
            pltpu.VMEM((n_chunks, CHUNK), jnp.int32),
            pltpu.VMEM((CHUNK, EMBED_DIM), jnp.float32),
            pltpu.SemaphoreType.DMA,
        ],
    )
    def k(table_hbm, idx_hbm, out_hbm, idx_v, rows_v, sem):
        wid = lax.axis_index("s") * 2 + lax.axis_index("c")
        base = wid * (n_chunks * CHUNK)
        pltpu.sync_copy(idx_hbm.at[wid], idx_v)

        def body(c, carry):
            pltpu.async_copy(table_hbm.at[idx_v.at[c]], rows_v, sem).wait()
            pltpu.sync_copy(rows_v, out_hbm.at[pl.ds(base + c * CHUNK, CHUNK)])
            return carry

        lax.fori_loop(0, n_chunks, body, 0)

    return k(weight, idx_grp)


def kernel(token_ids, weight):
    b, h = token_ids.shape
    total = b * h
    n_chunks = total // (NUM_WORKERS * CHUNK)
    idx_grp = token_ids.reshape(NUM_WORKERS, n_chunks, CHUNK)
    out = _sc_gather(weight, idx_grp, n_chunks)
    return out.reshape(b, h, EMBED_DIM)


# speedup vs baseline: 5.7350x; 1.1529x over previous
import functools

import jax
import jax.numpy as jnp
from jax import lax
from jax.experimental import pallas as pl
from jax.experimental.pallas import tpu as pltpu
from jax.experimental.pallas import tpu_sc as plsc

EMBED_DIM = 128
CHUNK = 128
NUM_WORKERS = 32


@functools.partial(jax.jit, static_argnames=("n_chunks",))
def _sc_gather(weight, idx_grp, n_chunks):
    batch = NUM_WORKERS * n_chunks * CHUNK
    mesh = plsc.VectorSubcoreMesh(core_axis_name="c", subcore_axis_name="s")

    @functools.partial(
        pl.kernel,
        mesh=mesh,
        out_type=jax.ShapeDtypeStruct((batch, EMBED_DIM), jnp.float32),
        scratch_types=[
            pltpu.VMEM((n_chunks, CHUNK), jnp.int32),
            pltpu.VMEM((2, CHUNK, EMBED_DIM), jnp.float32),
            pltpu.SemaphoreType.DMA,
            pltpu.SemaphoreType.DMA,
        ],
    )
    def k(table_hbm, idx_hbm, out_hbm, idx_v, rows_v, sem0, sem1):
        wid = lax.axis_index("s") * 2 + lax.axis_index("c")
        base = wid * (n_chunks * CHUNK)
        pltpu.sync_copy(idx_hbm.at[wid], idx_v)
        sems = (sem0, sem1)

        for b in range(2):
            pltpu.async_copy(table_hbm.at[idx_v.at[b]], rows_v.at[b], sems[b])

        def body(t, carry):
            for b in range(2):
                c = t * 2 + b
                pltpu.make_async_copy(
                    table_hbm.at[pl.ds(0, CHUNK)], rows_v.at[b], sems[b]
                ).wait()
                pltpu.sync_copy(
                    rows_v.at[b], out_hbm.at[pl.ds(base + c * CHUNK, CHUNK)]
                )

                @pl.when(c + 2 < n_chunks)
                def _():
                    pltpu.async_copy(
                        table_hbm.at[idx_v.at[c + 2]], rows_v.at[b], sems[b]
                    )

            return carry

        lax.fori_loop(0, n_chunks // 2, body, 0)

    return k(weight, idx_grp)


def kernel(token_ids, weight):
    b, h = token_ids.shape
    total = b * h
    n_chunks = total // (NUM_WORKERS * CHUNK)
    idx_grp = token_ids.reshape(NUM_WORKERS, n_chunks, CHUNK)
    out = _sc_gather(weight, idx_grp, n_chunks)
    return out.reshape(b, h, EMBED_DIM)


# speedup vs baseline: 5.7374x; 1.0004x over previous
import functools

import jax
import jax.numpy as jnp
from jax import lax
from jax.experimental import pallas as pl
from jax.experimental.pallas import tpu as pltpu
from jax.experimental.pallas import tpu_sc as plsc

EMBED_DIM = 128
CHUNK = 128
NUM_WORKERS = 32


@functools.partial(jax.jit, static_argnames=("n_chunks",))
def _sc_gather(weight, idx_grp, n_chunks):
    batch = NUM_WORKERS * n_chunks * CHUNK
    mesh = plsc.VectorSubcoreMesh(core_axis_name="c", subcore_axis_name="s")

    @functools.partial(
        pl.kernel,
        mesh=mesh,
        out_type=jax.ShapeDtypeStruct((batch, EMBED_DIM), jnp.float32),
        scratch_types=[
            pltpu.VMEM((n_chunks, CHUNK), jnp.int32),
            pltpu.VMEM((2, CHUNK, EMBED_DIM), jnp.float32),
            pltpu.SemaphoreType.DMA,
            pltpu.SemaphoreType.DMA,
            pltpu.SemaphoreType.DMA,
            pltpu.SemaphoreType.DMA,
        ],
    )
    def k(table_hbm, idx_hbm, out_hbm, idx_v, rows_v, sg0, sg1, ss0, ss1):
        wid = lax.axis_index("s") * 2 + lax.axis_index("c")
        base = wid * (n_chunks * CHUNK)
        pltpu.sync_copy(idx_hbm.at[wid], idx_v)
        sem_g = (sg0, sg1)
        sem_s = (ss0, ss1)

        pltpu.async_copy(table_hbm.at[idx_v.at[0]], rows_v.at[0], sem_g[0])

        def body(t, carry):
            for b in range(2):
                c = t * 2 + b
                nb = 1 - b

                @pl.when(c + 1 < n_chunks)
                def _():
                    @pl.when(c >= 1)
                    def _():
                        pltpu.make_async_copy(
                            rows_v.at[nb],
                            out_hbm.at[pl.ds(base, CHUNK)],
                            sem_s[nb],
                        ).wait()

                    pltpu.async_copy(
                        table_hbm.at[idx_v.at[c + 1]], rows_v.at[nb], sem_g[nb]
                    )

                pltpu.make_async_copy(
                    table_hbm.at[pl.ds(0, CHUNK)], rows_v.at[b], sem_g[b]
                ).wait()
                pltpu.async_copy(
                    rows_v.at[b], out_hbm.at[pl.ds(base + c * CHUNK, CHUNK)], sem_s[b]
                )

            return carry

        lax.fori_loop(0, n_chunks // 2, body, 0)

        for b in range(2):
            pltpu.make_async_copy(
                rows_v.at[b], out_hbm.at[pl.ds(base, CHUNK)], sem_s[b]
            ).wait()

    return k(weight, idx_grp)


def kernel(token_ids, weight):
    b, h = token_ids.shape
    total = b * h
    n_chunks = total // (NUM_WORKERS * CHUNK)
    idx_grp = token_ids.reshape(NUM_WORKERS, n_chunks, CHUNK)
    out = _sc_gather(weight, idx_grp, n_chunks)
    return out.reshape(b, h, EMBED_DIM)
